# baseline (speedup 1.0000x reference)
import functools

import jax
import jax.numpy as jnp
from jax import lax
from jax.experimental import pallas as pl
from jax.experimental.pallas import tpu as pltpu
from jax.experimental.pallas import tpu_sc as plsc

N_USERS = 4096
X_DIM = 64
K_SUBS = 20000
R_DIM = 3

NC = 2
NS = 16
NW = NC * NS
U_W = N_USERS // NW
NV = U_W // 16
KC = 400
NCHUNK = K_SUBS // KC

_mesh = plsc.VectorSubcoreMesh(core_axis_name="c", subcore_axis_name="s")


@functools.partial(
    pl.kernel,
    out_type=jax.ShapeDtypeStruct((R_DIM, N_USERS), jnp.float32),
    mesh=_mesh,
    scratch_types=[
        pltpu.VMEM((KC, U_W), jnp.float32),
        pltpu.VMEM((KC, 16), jnp.float32),
        pltpu.VMEM((R_DIM, U_W), jnp.float32),
    ],
)
def _sc_agg(st_hbm, r_hbm, out_hbm, sbuf, rbuf, obuf):
    wid = lax.axis_index("s") * NC + lax.axis_index("c")
    ubase = wid * U_W

    def chunk_body(c, accs):
        k0 = c * KC
        pltpu.sync_copy(st_hbm.at[pl.ds(k0, KC), pl.ds(ubase, U_W)], sbuf)
        pltpu.sync_copy(r_hbm.at[pl.ds(k0, KC), :], rbuf)

        def k_body(k, a):
            new = list(a)
            rv = rbuf[k, :]
            for v in range(NV):
                sv = sbuf[k, pl.ds(v * 16, 16)]
                for j in range(R_DIM):
                    new[j * NV + v] = a[j * NV + v] + sv * rv[j]
            return tuple(new)

        return lax.fori_loop(0, KC, k_body, accs)

    zero = jnp.zeros((16,), jnp.float32)
    accs = lax.fori_loop(0, NCHUNK, chunk_body, (zero,) * (R_DIM * NV))

    for j in range(R_DIM):
        for v in range(NV):
            obuf[j, pl.ds(v * 16, 16)] = accs[j * NV + v]
    pltpu.sync_copy(obuf, out_hbm.at[:, pl.ds(ubase, U_W)])


def kernel(x, S, R):
    r_pad = jnp.pad(R, ((0, 0), (0, 16 - R_DIM)))
    agg_t = _sc_agg(S.T, r_pad)
    return jnp.concatenate([x, agg_t.T], axis=1)


# speedup vs baseline: 4.1383x; 4.1383x over previous
import jax
import jax.numpy as jnp
from jax.experimental import pallas as pl
from jax.experimental.pallas import tpu as pltpu

N_USERS = 4096
X_DIM = 64
K_SUBS = 20000
R_DIM = 3

BK = 1000
NK = K_SUBS // BK


def _body(st_ref, r_ref, o_ref, acc_ref):
    k = pl.program_id(0)

    @pl.when(k == 0)
    def _init():
        acc_ref[...] = jnp.zeros_like(acc_ref)

    r_blk = r_ref[pl.ds(pl.multiple_of(k * BK, 8), BK), :]
    acc_ref[...] += jax.lax.dot_general(
        st_ref[...],
        r_blk,
        (((0,), (0,)), ((), ())),
        preferred_element_type=jnp.float32,
    )

    @pl.when(k == NK - 1)
    def _fin():
        o_ref[...] = acc_ref[...]


def kernel(x, S, R):
    agg = pl.pallas_call(
        _body,
        grid=(NK,),
        in_specs=[
            pl.BlockSpec((BK, N_USERS), lambda k: (k, 0)),
            pl.BlockSpec((K_SUBS, R_DIM), lambda k: (0, 0)),
        ],
        out_specs=pl.BlockSpec((N_USERS, R_DIM), lambda k: (0, 0)),
        out_shape=jax.ShapeDtypeStruct((N_USERS, R_DIM), jnp.float32),
        scratch_shapes=[pltpu.VMEM((N_USERS, R_DIM), jnp.float32)],
        compiler_params=pltpu.CompilerParams(
            dimension_semantics=("arbitrary",),
        ),
    )(S.T, R)
    return jnp.concatenate([x, agg], axis=1)
